# baseline (speedup 1.0000x reference)
import functools

import jax
import jax.numpy as jnp
from jax import lax
from jax.experimental import pallas as pl
from jax.experimental.pallas import tpu as pltpu
from jax.experimental.pallas import tpu_sc as plsc

B = 4096
L = 50
D = 64
H = 256
C = 10

NC = 2
NS = 16
NW = NC * NS
ROWS_PER_W = B // NW
ROWS_PER_CHUNK = 2
IDX_PER_CHUNK = ROWS_PER_CHUNK * L
CPW = ROWS_PER_W // ROWS_PER_CHUNK
NCHUNKS = B // ROWS_PER_CHUNK
LANES = 16
G = D // LANES

UNROLL = 5


def _make_pool_body(nt, NBUF):
  def body(*args):
    xs = args[0:nt]
    tab = args[nt]
    out_hbm = args[nt + 1]
    idx_v, rows_v, out_v = args[nt + 2:nt + 5]
    sems = tuple(args[nt + 5 + NBUF * t: nt + 5 + NBUF * (t + 1)]
                 for t in range(nt))

    c = lax.axis_index("c")
    s = lax.axis_index("s")
    wid = s * NC + c
    iota = lax.iota(jnp.int32, LANES)

    for t in range(nt):
      pltpu.sync_copy(xs[t].at[pl.ds(wid * CPW, CPW)], idx_v.at[t])

    def start(t, i, p):
      pltpu.async_copy(tab.at[idx_v.at[t, i]], rows_v.at[t, p], sems[t][p])

    def accum(t, i, p):
      def acc_body(j, accs):
        new = list(accs)
        for u in range(UNROLL):
          for r in range(ROWS_PER_CHUNK):
            for g in range(G):
              new[r * G + g] = (
                  new[r * G + g]
                  + rows_v[t, p, r * L + j * UNROLL + u,
                           pl.ds(g * LANES, LANES)])
        return tuple(new)

      accs = lax.fori_loop(
          0, L // UNROLL, acc_body,
          tuple(jnp.zeros((LANES,), jnp.float32)
                for _ in range(ROWS_PER_CHUNK * G)))
      for r in range(ROWS_PER_CHUNK):
        col = jnp.full((LANES,), i * ROWS_PER_CHUNK + r, jnp.int32)
        for g in range(G):
          plsc.store_scatter(out_v, [iota + (t * D + g * LANES), col],
                             accs[r * G + g] * (1.0 / L))

    for b in range(NBUF):
      for t in range(nt):
        start(t, b, b)

    def step(k, _):
      c0 = NBUF * k
      for b in range(NBUF):
        for t in range(nt):
          pltpu.make_async_copy(tab.at[idx_v.at[t, c0 + b]],
                                rows_v.at[t, b], sems[t][b]).wait()
          accum(t, c0 + b, b)

          @pl.when(k < CPW // NBUF - 1)
          def _(t=t, b=b):
            start(t, c0 + b + NBUF, b)
      return 0

    lax.fori_loop(0, CPW // NBUF, step, 0)

    pltpu.sync_copy(out_v, out_hbm.at[:, pl.ds(wid * ROWS_PER_W, ROWS_PER_W)])

  return body


def _make_pool(nt, NBUF):
  return functools.partial(
      pl.kernel,
      out_type=jax.ShapeDtypeStruct((nt * D, B), jnp.float32),
      mesh=plsc.VectorSubcoreMesh(
          core_axis_name="c", subcore_axis_name="s", num_cores=NC),
      scratch_types=[
          pltpu.VMEM((nt, CPW, IDX_PER_CHUNK), jnp.int32),
          pltpu.VMEM((nt, NBUF, IDX_PER_CHUNK, D), jnp.float32),
          pltpu.VMEM((nt * D, ROWS_PER_W), jnp.float32),
      ] + [pltpu.SemaphoreType.DMA] * (NBUF * nt),
      compiler_params=pltpu.CompilerParams(
          use_tc_tiling_on_sc=False, needs_layout_passes=False),
  )(_make_pool_body(nt, NBUF))


_sc_pool_ab = _make_pool(2, 4)
_sc_pool_c = _make_pool(1, 8)


XB = 8192


def _xpose2_body(at_ref, bt_ref, o_ref):
  o_ref[:, 0:D] = jnp.transpose(at_ref[...])
  o_ref[:, D:2 * D] = jnp.transpose(bt_ref[...])


def _xpose1_body(at_ref, o_ref):
  o_ref[:, 0:D] = jnp.transpose(at_ref[...])


def _tab_xpose2(tab_a, tab_b):
  vn = tab_a.shape[0]
  grid = (vn + XB - 1) // XB
  out = pl.pallas_call(
      _xpose2_body,
      grid=(grid,),
      in_specs=[pl.BlockSpec((D, XB), lambda i: (0, i)),
                pl.BlockSpec((D, XB), lambda i: (0, i))],
      out_specs=pl.BlockSpec((XB, 2 * D), lambda i: (i, 0)),
      out_shape=jax.ShapeDtypeStruct((vn, 2 * D), jnp.float32),
  )(jnp.transpose(tab_a), jnp.transpose(tab_b))
  return out.reshape(2 * vn, D)


def _tab_xpose1(tab):
  vn = tab.shape[0]
  grid = (vn + XB - 1) // XB
  out = pl.pallas_call(
      _xpose1_body,
      grid=(grid,),
      in_specs=[pl.BlockSpec((D, XB), lambda i: (0, i))],
      out_specs=pl.BlockSpec((XB, 2 * D), lambda i: (i, 0)),
      out_shape=jax.ShapeDtypeStruct((vn, 2 * D), jnp.float32),
  )(jnp.transpose(tab))
  return out.reshape(2 * vn, D)


CPAD = 16
BBLK = 1024


def _mlp_body(pab_ref, pc_ref, w1_ref, b1_ref, w2_ref, b2_ref, o_ref):
  w1t = w1_ref[...]
  h = lax.dot_general(w1t[0:2 * D, :], pab_ref[...], (((0,), (0,)), ((), ())),
                      preferred_element_type=jnp.float32)
  h = h + lax.dot_general(w1t[2 * D:3 * D, :], pc_ref[...],
                          (((0,), (0,)), ((), ())),
                          preferred_element_type=jnp.float32)
  h = jnp.maximum(h + b1_ref[...], 0.0)
  o = lax.dot_general(w2_ref[...], h, (((1,), (0,)), ((), ())),
                      preferred_element_type=jnp.float32)
  o_ref[...] = o + b2_ref[...]


def _mlp_t(pab, pc, W1, b1c, W2p, b2c):
  return pl.pallas_call(
      _mlp_body,
      grid=(B // BBLK,),
      in_specs=[
          pl.BlockSpec((2 * D, BBLK), lambda i: (0, i)),
          pl.BlockSpec((D, BBLK), lambda i: (0, i)),
          pl.BlockSpec((3 * D, H), lambda i: (0, 0)),
          pl.BlockSpec((H, 1), lambda i: (0, 0)),
          pl.BlockSpec((CPAD, H), lambda i: (0, 0)),
          pl.BlockSpec((CPAD, 1), lambda i: (0, 0)),
      ],
      out_specs=pl.BlockSpec((CPAD, BBLK), lambda i: (0, i)),
      out_shape=jax.ShapeDtypeStruct((CPAD, B), jnp.float32),
  )(pab, pc, W1, b1c, W2p, b2c)


def kernel(x0, x2, x3, emb_word, emb_bigram, emb_trigram, W1, b1, W2, b2):
  x0r = (x0.astype(jnp.int32) * 2).reshape(NCHUNKS, IDX_PER_CHUNK)
  x2r = (x2.astype(jnp.int32) * 2 + 1).reshape(NCHUNKS, IDX_PER_CHUNK)
  x3r = (x3.astype(jnp.int32) * 2).reshape(NCHUNKS, IDX_PER_CHUNK)
  tab_ab = _tab_xpose2(emb_word, emb_bigram)
  tab_c = _tab_xpose1(emb_trigram)
  pab = _sc_pool_ab(x0r, x2r, tab_ab)
  pc = _sc_pool_c(x3r, tab_c)
  W2p = jnp.zeros((CPAD, H), jnp.float32).at[:C].set(W2)
  b2c = jnp.zeros((CPAD, 1), jnp.float32).at[:C, 0].set(b2)
  oT = _mlp_t(pab, pc, jnp.transpose(W1), b1.reshape(H, 1), W2p, b2c)
  return jnp.transpose(oT[:C, :])


# speedup vs baseline: 1.0127x; 1.0127x over previous
import functools

import jax
import jax.numpy as jnp
from jax import lax
from jax.experimental import pallas as pl
from jax.experimental.pallas import tpu as pltpu
from jax.experimental.pallas import tpu_sc as plsc

B = 4096
L = 50
D = 64
H = 256
C = 10

NC = 2
NS = 16
NW = NC * NS
ROWS_PER_W = B // NW
ROWS_PER_CHUNK = 2
IDX_PER_CHUNK = ROWS_PER_CHUNK * L
CPW = ROWS_PER_W // ROWS_PER_CHUNK
NCHUNKS = B // ROWS_PER_CHUNK
LANES = 16
G = D // LANES

UNROLL = 5


def _make_pool_body(nf, NBUF, cpw):
  rows_per_chunk = IDX_PER_CHUNK // L

  def body(x_hbm, tab, out_hbm, idx_v, rows_v, out_v, *sems):
    c = lax.axis_index("c")
    s = lax.axis_index("s")
    wid = s * NC + c
    iota = lax.iota(jnp.int32, LANES)

    pltpu.sync_copy(x_hbm.at[pl.ds(wid * cpw, cpw)], idx_v)

    def start(i, p):
      pltpu.async_copy(tab.at[idx_v.at[i]], rows_v.at[p], sems[p])

    def accum(i, p):
      def acc_body(j, accs):
        new = list(accs)
        for u in range(UNROLL):
          for r in range(rows_per_chunk):
            for g in range(G):
              new[r * G + g] = (
                  new[r * G + g]
                  + rows_v[p, r * L + j * UNROLL + u,
                           pl.ds(g * LANES, LANES)])
        return tuple(new)

      accs = lax.fori_loop(
          0, L // UNROLL, acc_body,
          tuple(jnp.zeros((LANES,), jnp.float32)
                for _ in range(rows_per_chunk * G)))
      for r in range(rows_per_chunk):
        if nf == 2:
          col = jnp.full((LANES,), i, jnp.int32)
          frow = r * D
        else:
          col = jnp.full((LANES,), i * rows_per_chunk + r, jnp.int32)
          frow = 0
        for g in range(G):
          plsc.store_scatter(out_v, [iota + (frow + g * LANES), col],
                             accs[r * G + g] * (1.0 / L))

    for b in range(NBUF):
      start(b, b)

    def step(k, _):
      c0 = NBUF * k
      for b in range(NBUF):
        pltpu.make_async_copy(tab.at[idx_v.at[c0 + b]],
                              rows_v.at[b], sems[b]).wait()
        accum(c0 + b, b)

        @pl.when(k < cpw // NBUF - 1)
        def _(b=b):
          start(c0 + b + NBUF, b)
      return 0

    lax.fori_loop(0, cpw // NBUF, step, 0)

    pltpu.sync_copy(out_v, out_hbm.at[:, pl.ds(wid * ROWS_PER_W, ROWS_PER_W)])

  return body


def _make_pool(nf, NBUF):
  cpw = ROWS_PER_W if nf == 2 else CPW
  return functools.partial(
      pl.kernel,
      out_type=jax.ShapeDtypeStruct((nf * D, B), jnp.float32),
      mesh=plsc.VectorSubcoreMesh(
          core_axis_name="c", subcore_axis_name="s", num_cores=NC),
      scratch_types=[
          pltpu.VMEM((cpw, IDX_PER_CHUNK), jnp.int32),
          pltpu.VMEM((NBUF, IDX_PER_CHUNK, D), jnp.float32),
          pltpu.VMEM((nf * D, ROWS_PER_W), jnp.float32),
      ] + [pltpu.SemaphoreType.DMA] * NBUF,
      compiler_params=pltpu.CompilerParams(
          use_tc_tiling_on_sc=False, needs_layout_passes=False),
  )(_make_pool_body(nf, NBUF, cpw))


_sc_pool_ab = _make_pool(2, 8)
_sc_pool_c = _make_pool(1, 8)


XB = 8192


def _xpose2_body(at_ref, bt_ref, o_ref):
  o_ref[:, 0:D] = jnp.transpose(at_ref[...])
  o_ref[:, D:2 * D] = jnp.transpose(bt_ref[...])


def _xpose1_body(at_ref, o_ref):
  o_ref[:, 0:D] = jnp.transpose(at_ref[...])


def _tab_xpose2(tab_a, tab_b):
  vn = tab_a.shape[0]
  grid = (vn + XB - 1) // XB
  out = pl.pallas_call(
      _xpose2_body,
      grid=(grid,),
      in_specs=[pl.BlockSpec((D, XB), lambda i: (0, i)),
                pl.BlockSpec((D, XB), lambda i: (0, i))],
      out_specs=pl.BlockSpec((XB, 2 * D), lambda i: (i, 0)),
      out_shape=jax.ShapeDtypeStruct((vn, 2 * D), jnp.float32),
  )(jnp.transpose(tab_a), jnp.transpose(tab_b))
  return out.reshape(2 * vn, D)


def _tab_xpose1(tab):
  vn = tab.shape[0]
  grid = (vn + XB - 1) // XB
  out = pl.pallas_call(
      _xpose1_body,
      grid=(grid,),
      in_specs=[pl.BlockSpec((D, XB), lambda i: (0, i))],
      out_specs=pl.BlockSpec((XB, 2 * D), lambda i: (i, 0)),
      out_shape=jax.ShapeDtypeStruct((vn, 2 * D), jnp.float32),
  )(jnp.transpose(tab))
  return out.reshape(2 * vn, D)


CPAD = 16
BBLK = 1024


def _mlp_body(pab_ref, pc_ref, w1_ref, b1_ref, w2_ref, b2_ref, o_ref):
  w1t = w1_ref[...]
  h = lax.dot_general(w1t[0:2 * D, :], pab_ref[...], (((0,), (0,)), ((), ())),
                      preferred_element_type=jnp.float32)
  h = h + lax.dot_general(w1t[2 * D:3 * D, :], pc_ref[...],
                          (((0,), (0,)), ((), ())),
                          preferred_element_type=jnp.float32)
  h = jnp.maximum(h + b1_ref[...], 0.0)
  o = lax.dot_general(w2_ref[...], h, (((1,), (0,)), ((), ())),
                      preferred_element_type=jnp.float32)
  o_ref[...] = o + b2_ref[...]


def _mlp_t(pab, pc, W1, b1c, W2p, b2c):
  return pl.pallas_call(
      _mlp_body,
      grid=(B // BBLK,),
      in_specs=[
          pl.BlockSpec((2 * D, BBLK), lambda i: (0, i)),
          pl.BlockSpec((D, BBLK), lambda i: (0, i)),
          pl.BlockSpec((3 * D, H), lambda i: (0, 0)),
          pl.BlockSpec((H, 1), lambda i: (0, 0)),
          pl.BlockSpec((CPAD, H), lambda i: (0, 0)),
          pl.BlockSpec((CPAD, 1), lambda i: (0, 0)),
      ],
      out_specs=pl.BlockSpec((CPAD, BBLK), lambda i: (0, i)),
      out_shape=jax.ShapeDtypeStruct((CPAD, B), jnp.float32),
  )(pab, pc, W1, b1c, W2p, b2c)


def kernel(x0, x2, x3, emb_word, emb_bigram, emb_trigram, W1, b1, W2, b2):
  x_ab = jnp.concatenate(
      [x0.astype(jnp.int32) * 2, x2.astype(jnp.int32) * 2 + 1], axis=1)
  x3r = (x3.astype(jnp.int32) * 2).reshape(NCHUNKS, IDX_PER_CHUNK)
  tab_ab = _tab_xpose2(emb_word, emb_bigram)
  tab_c = _tab_xpose1(emb_trigram)
  pab = _sc_pool_ab(x_ab, tab_ab)
  pc = _sc_pool_c(x3r, tab_c)
  W2p = jnp.zeros((CPAD, H), jnp.float32).at[:C].set(W2)
  b2c = jnp.zeros((CPAD, 1), jnp.float32).at[:C, 0].set(b2)
  oT = _mlp_t(pab, pc, jnp.transpose(W1), b1.reshape(H, 1), W2p, b2c)
  return jnp.transpose(oT[:C, :])
